# baseline (speedup 1.0000x reference)
import functools

import jax
import jax.numpy as jnp
from jax import lax
from jax.experimental import pallas as pl
from jax.experimental.pallas import tpu as pltpu
from jax.experimental.pallas import tpu_sc as plsc

B = 16384
D = 512
SEG = 64
NSEG = 8
V = 100000

NC = 2
NS = 16
NW = NC * NS

B_PER_W = B // NW
CHUNK = 64
N_CHUNKS = B_PER_W // CHUNK
N_PAIRS = N_CHUNKS // 2

LN_BLOCK = 1024


def _gather_body(idx_hbm, table_hbm, out_hbm, idx_v, buf0, buf1,
                 in0, in1, out0, out1):
    wid = lax.axis_index("c") * NS + lax.axis_index("s")
    base = wid * B_PER_W

    pltpu.sync_copy(idx_hbm.at[pl.ds(base, B_PER_W)], idx_v)

    def start_gather(c, buf, sem):
        pltpu.make_async_copy(
            table_hbm.at[idx_v.at[pl.ds(c * CHUNK, CHUNK)]], buf, sem
        ).start()

    def wait_gather(c, buf, sem):
        pltpu.make_async_copy(
            table_hbm.at[idx_v.at[pl.ds(c * CHUNK, CHUNK)]], buf, sem
        ).wait()

    def start_out(c, buf, sem):
        pltpu.make_async_copy(
            buf, out_hbm.at[pl.ds(base + c * CHUNK, CHUNK)], sem
        ).start()

    def wait_out(c, buf, sem):
        pltpu.make_async_copy(
            buf, out_hbm.at[pl.ds(base + c * CHUNK, CHUNK)], sem
        ).wait()

    start_gather(0, buf0, in0)

    def pair_step(t, _):
        c0 = 2 * t
        c1 = c0 + 1

        @pl.when(t > 0)
        def _():
            wait_out(c0 - 1, buf1, out1)

        start_gather(c1, buf1, in1)
        wait_gather(c0, buf0, in0)
        start_out(c0, buf0, out0)

        wait_gather(c1, buf1, in1)
        start_out(c1, buf1, out1)

        @pl.when(t < N_PAIRS - 1)
        def _():
            wait_out(c0, buf0, out0)
            start_gather(c0 + 2, buf0, in0)

        return 0

    lax.fori_loop(0, N_PAIRS, pair_step, 0)
    wait_out(N_CHUNKS - 2, buf0, out0)
    wait_out(N_CHUNKS - 1, buf1, out1)


def _ln_body(x_ref, gamma_ref, beta_ref, out_ref):
    x = x_ref[...].reshape(LN_BLOCK, NSEG, SEG)
    mean = jnp.mean(x, axis=-1, keepdims=True)
    var = jnp.mean(jnp.square(x - mean), axis=-1, keepdims=True)
    xn = (x - mean) * lax.rsqrt(var + jnp.float32(1e-5))
    out_ref[...] = xn * gamma_ref[...] + beta_ref[...]


@jax.jit
def _encode(prompt_idx, table, gamma, beta):
    mesh = plsc.VectorSubcoreMesh(core_axis_name="c", subcore_axis_name="s")
    gathered = jnp.take(table, prompt_idx, axis=0)

    return pl.pallas_call(
        _ln_body,
        grid=(B // LN_BLOCK,),
        in_specs=[
            pl.BlockSpec((LN_BLOCK, D), lambda i: (i, 0)),
            pl.BlockSpec((SEG,), lambda i: (0,)),
            pl.BlockSpec((SEG,), lambda i: (0,)),
        ],
        out_specs=pl.BlockSpec((LN_BLOCK, NSEG, SEG), lambda i: (i, 0, 0)),
        out_shape=jax.ShapeDtypeStruct((B, NSEG, SEG), jnp.float32),
    )(gathered, gamma, beta)


def kernel(prompt_idx, table, gamma, beta):
    return _encode(prompt_idx, table, gamma, beta)
